# baseline (speedup 1.0000x reference)
import functools

import jax
import jax.numpy as jnp
import numpy as np
from jax import lax
from jax.experimental import pallas as pl
from jax.experimental.pallas import tpu as pltpu
from jax.experimental.pallas import tpu_sc as plsc

_BINS = 10
_ALPHA = 0.75
_EDGES = [np.float32(i) / np.float32(10.0) for i in range(_BINS + 1)]

_NW = 32
_CHUNK = 8192


def _elementwise(x, t):
    ax = jnp.abs(x)
    en = jnp.exp(-ax)
    l1p = _log1p_poly(en)
    loss = jnp.maximum(x, 0.0) - x * t + l1p
    p1 = 1.0 / (1.0 + en)
    pred = jnp.where(x >= 0.0, p1, en * p1)
    g = jnp.abs(pred - t)
    return loss, g


def _log1p_poly(y):
    z = y / (2.0 + y)
    z2 = z * z
    p = np.float32(1 / 7) + z2 * 0.0
    p = np.float32(1 / 5) + z2 * p
    p = np.float32(1 / 3) + z2 * p
    return 2.0 * z * (1.0 + z2 * p)


def _accumulate(accs, loss, g):
    new = list(accs)
    new[0] = new[0] + loss
    for i in range(1, _BINS + 1):
        m = g >= _EDGES[i]
        if i < _BINS:
            new[i] = new[i] + jnp.where(m, loss, 0.0)
        new[9 + i] = new[9 + i] + jnp.where(m, 1.0, 0.0)
    return tuple(new)


def _ghm_tc_body(x_ref, t_ref, out_ref, *, blk_rows):
    def chunk(r0, accs):
        x = x_ref[pl.ds(r0, 8), :]
        t = t_ref[pl.ds(r0, 8), :]
        ax = jnp.abs(x)
        en = jnp.exp(-ax)
        loss = jnp.maximum(x, 0.0) - x * t + jnp.log1p(en)
        p1 = 1.0 / (1.0 + en)
        pred = jnp.where(x >= 0.0, p1, en * p1)
        g = jnp.abs(pred - t)
        return _accumulate(accs, loss, g)

    zero = jnp.zeros((8, 128), jnp.float32)
    accs = (zero,) * 20
    for c in range(blk_rows // 8):
        accs = chunk(8 * c, accs)
    for j in range(20):
        out_ref[0, 8 * j:8 * j + 8, :] = accs[j]


def _tc_partials(xr, tr, n_tc):
    cols = 128
    blk_rows = 1024
    grid = n_tc // cols // blk_rows
    out = pl.pallas_call(
        functools.partial(_ghm_tc_body, blk_rows=blk_rows),
        grid=(grid,),
        in_specs=[
            pl.BlockSpec((blk_rows, cols), lambda i: (i, 0)),
            pl.BlockSpec((blk_rows, cols), lambda i: (i, 0)),
        ],
        out_specs=pl.BlockSpec((1, 160, 128), lambda i: (i, 0, 0)),
        out_shape=jax.ShapeDtypeStruct((grid, 160, 128), jnp.float32),
        compiler_params=pltpu.CompilerParams(
            dimension_semantics=("parallel",)),
    )(xr, tr)
    return jnp.sum(out.reshape(grid, 20, 8 * 128), axis=(0, 2))


def _make_sc_partials(n_tc, n_sc):
    e_w = n_sc // _NW
    nchunks = e_w // _CHUNK
    mesh = plsc.VectorSubcoreMesh(core_axis_name="c", subcore_axis_name="s")

    @functools.partial(
        pl.kernel,
        mesh=mesh,
        out_type=jax.ShapeDtypeStruct((_NW, 512), jnp.float32),
        scratch_types=[
            pltpu.VMEM((2, _CHUNK), jnp.float32),
            pltpu.VMEM((2, _CHUNK), jnp.float32),
            pltpu.VMEM((512,), jnp.float32),
            pltpu.SemaphoreType.DMA,
            pltpu.SemaphoreType.DMA,
            pltpu.SemaphoreType.DMA,
            pltpu.SemaphoreType.DMA,
        ],
    )
    def sc_part(x_hbm, t_hbm, out_hbm, xb, tb, hist, sa0, sa1, sb0, sb1):
        wid = lax.axis_index("s") * 2 + lax.axis_index("c")
        base = n_tc + wid * e_w
        sems = ((sa0, sb0), (sa1, sb1))
        lanes = lax.iota(jnp.int32, 16)
        ones = jnp.ones((16,), jnp.float32)

        for j in range(32):
            hist[pl.ds(16 * j, 16)] = jnp.zeros((16,), jnp.float32)

        def issue(ci, s):
            hx = pltpu.async_copy(
                x_hbm.at[pl.ds(base + ci * _CHUNK, _CHUNK)], xb.at[s],
                sems[s][0])
            ht = pltpu.async_copy(
                t_hbm.at[pl.ds(base + ci * _CHUNK, _CHUNK)], tb.at[s],
                sems[s][1])
            return hx, ht

        def compute(s, accs):
            def body(i, accs):
                for u in range(4):
                    x = xb[s, pl.ds(64 * i + 16 * u, 16)]
                    t = tb[s, pl.ds(64 * i + 16 * u, 16)]
                    loss, g = _elementwise(x, t)
                    accs = _accumulate(accs, loss, g)
                return accs

            return lax.fori_loop(0, _CHUNK // 64, body, accs)

        zero = jnp.zeros((16,), jnp.float32)
        accs = (zero,) * 20
        pending = issue(0, 0)
        for ci in range(nchunks):
            s = ci % 2
            if ci + 1 < nchunks:
                nxt = issue(ci + 1, (ci + 1) % 2)
            pending[0].wait()
            pending[1].wait()
            accs = compute(s, accs)
            if ci + 1 < nchunks:
                pending = nxt
        for j in range(20):
            hist[pl.ds(16 * j, 16)] = accs[j]
        pltpu.sync_copy(hist, out_hbm.at[wid])

    return sc_part


def kernel(x, target):
    n = x.size
    n_sc = 4 * 1024 * 1024
    n_tc = n - n_sc
    cols = 128

    xr = x.reshape(n // cols, cols)
    tr = target.reshape(n // cols, cols)

    sums_tc = _tc_partials(xr, tr, n_tc)
    out_sc = _make_sc_partials(n_tc, n_sc)(x, target)
    h = out_sc.reshape(_NW, 32, 16)
    sums = sums_tc + jnp.sum(h[:, 0:20, :], axis=(0, 2))

    t_j = sums[0:_BINS]
    c_i = sums[_BINS:2 * _BINS]
    nf = jnp.float32(n)
    tot = jnp.concatenate([jnp.array([nf], jnp.float32), c_i[:-1]]) - c_i
    loss_sum = t_j - jnp.concatenate([t_j[1:], jnp.zeros((1,), jnp.float32)])
    w = jnp.clip(tot, 1.0, None) ** jnp.float32(-_ALPHA)
    return jnp.sum(loss_sum * w) / nf


# speedup vs baseline: 2.6676x; 2.6676x over previous
import functools

import jax
import jax.numpy as jnp
import numpy as np
from jax import lax
from jax.experimental import pallas as pl
from jax.experimental.pallas import tpu as pltpu
from jax.experimental.pallas import tpu_sc as plsc

_BINS = 10
_ALPHA = 0.75
_EDGES = [np.float32(i) / np.float32(10.0) for i in range(_BINS + 1)]

_NW = 32
_CHUNK = 8192


def _elementwise(x, t):
    ax = jnp.abs(x)
    en = jnp.exp(-ax)
    l1p = _log1p_poly(en)
    loss = jnp.maximum(x, 0.0) - x * t + l1p
    p1 = 1.0 / (1.0 + en)
    pred = jnp.where(x >= 0.0, p1, en * p1)
    g = jnp.abs(pred - t)
    return loss, g


def _log1p_poly(y):
    z = y / (2.0 + y)
    z2 = z * z
    p = np.float32(1 / 7) + z2 * 0.0
    p = np.float32(1 / 5) + z2 * p
    p = np.float32(1 / 3) + z2 * p
    return 2.0 * z * (1.0 + z2 * p)


def _accumulate(accs, loss, g):
    new = list(accs)
    new[0] = new[0] + loss
    for i in range(1, _BINS + 1):
        m = g >= _EDGES[i]
        if i < _BINS:
            new[i] = new[i] + jnp.where(m, loss, 0.0)
        new[9 + i] = new[9 + i] + jnp.where(m, 1.0, 0.0)
    return tuple(new)


def _ghm_tc_body(x_ref, t_ref, out_ref, *, blk_rows):
    def chunk(r0, accs):
        x = x_ref[pl.ds(r0, 8), :]
        t = t_ref[pl.ds(r0, 8), :]
        ax = jnp.abs(x)
        en = jnp.exp(-ax)
        loss = jnp.maximum(x, 0.0) - x * t + jnp.log1p(en)
        p1 = 1.0 / (1.0 + en)
        pred = jnp.where(x >= 0.0, p1, en * p1)
        g = jnp.abs(pred - t)
        return _accumulate(accs, loss, g)

    zero = jnp.zeros((8, 128), jnp.float32)
    accs = (zero,) * 20
    for c in range(blk_rows // 8):
        accs = chunk(8 * c, accs)
    for j in range(20):
        out_ref[0, 8 * j:8 * j + 8, :] = accs[j]


def _tc_partials(xr, tr, n_tc):
    cols = 128
    blk_rows = 1024
    grid = n_tc // cols // blk_rows
    out = pl.pallas_call(
        functools.partial(_ghm_tc_body, blk_rows=blk_rows),
        grid=(grid,),
        in_specs=[
            pl.BlockSpec((blk_rows, cols), lambda i: (i, 0)),
            pl.BlockSpec((blk_rows, cols), lambda i: (i, 0)),
        ],
        out_specs=pl.BlockSpec((1, 160, 128), lambda i: (i, 0, 0)),
        out_shape=jax.ShapeDtypeStruct((grid, 160, 128), jnp.float32),
        compiler_params=pltpu.CompilerParams(
            dimension_semantics=("parallel",)),
    )(xr, tr)
    return jnp.sum(out.reshape(grid, 20, 8 * 128), axis=(0, 2))


def _make_sc_partials(n_tc, n_sc):
    e_w = n_sc // _NW
    nchunks = e_w // _CHUNK
    mesh = plsc.VectorSubcoreMesh(core_axis_name="c", subcore_axis_name="s")

    @functools.partial(
        pl.kernel,
        mesh=mesh,
        out_type=jax.ShapeDtypeStruct((_NW, 512), jnp.float32),
        scratch_types=[
            pltpu.VMEM((2, _CHUNK), jnp.float32),
            pltpu.VMEM((2, _CHUNK), jnp.float32),
            pltpu.VMEM((512,), jnp.float32),
            pltpu.SemaphoreType.DMA,
            pltpu.SemaphoreType.DMA,
            pltpu.SemaphoreType.DMA,
            pltpu.SemaphoreType.DMA,
        ],
    )
    def sc_part(x_hbm, t_hbm, out_hbm, xb, tb, hist, sa0, sa1, sb0, sb1):
        wid = lax.axis_index("s") * 2 + lax.axis_index("c")
        base = n_tc + wid * e_w
        sems = ((sa0, sb0), (sa1, sb1))
        lanes = lax.iota(jnp.int32, 16)
        ones = jnp.ones((16,), jnp.float32)

        for j in range(32):
            hist[pl.ds(16 * j, 16)] = jnp.zeros((16,), jnp.float32)

        def issue(ci, s):
            hx = pltpu.async_copy(
                x_hbm.at[pl.ds(base + ci * _CHUNK, _CHUNK)], xb.at[s],
                sems[s][0])
            ht = pltpu.async_copy(
                t_hbm.at[pl.ds(base + ci * _CHUNK, _CHUNK)], tb.at[s],
                sems[s][1])
            return hx, ht

        def compute(s, accs):
            def body(i, accs):
                x = xb[s, pl.ds(16 * i, 16)]
                t = tb[s, pl.ds(16 * i, 16)]
                loss, g = _elementwise(x, t)
                return _accumulate(accs, loss, g)

            return lax.fori_loop(0, _CHUNK // 16, body, accs)

        zero = jnp.zeros((16,), jnp.float32)
        accs = (zero,) * 20
        pending = issue(0, 0)
        for ci in range(nchunks):
            s = ci % 2
            if ci + 1 < nchunks:
                nxt = issue(ci + 1, (ci + 1) % 2)
            pending[0].wait()
            pending[1].wait()
            accs = compute(s, accs)
            if ci + 1 < nchunks:
                pending = nxt
        for j in range(20):
            hist[pl.ds(16 * j, 16)] = accs[j]
        pltpu.sync_copy(hist, out_hbm.at[wid])

    return sc_part


def kernel(x, target):
    n = x.size
    n_sc = 4 * 1024 * 1024
    n_tc = n - n_sc
    cols = 128

    xr = x.reshape(n // cols, cols)
    tr = target.reshape(n // cols, cols)

    sums_tc = _tc_partials(xr, tr, n_tc)
    out_sc = _make_sc_partials(n_tc, n_sc)(x, target)
    h = out_sc.reshape(_NW, 32, 16)
    sums = sums_tc + jnp.sum(h[:, 0:20, :], axis=(0, 2))

    t_j = sums[0:_BINS]
    c_i = sums[_BINS:2 * _BINS]
    nf = jnp.float32(n)
    tot = jnp.concatenate([jnp.array([nf], jnp.float32), c_i[:-1]]) - c_i
    loss_sum = t_j - jnp.concatenate([t_j[1:], jnp.zeros((1,), jnp.float32)])
    w = jnp.clip(tot, 1.0, None) ** jnp.float32(-_ALPHA)
    return jnp.sum(loss_sum * w) / nf
